# Initial kernel scaffold; baseline (speedup 1.0000x reference)
import jax
import jax.numpy as jnp
from jax.experimental import pallas as pl


def kernel(x, edge_index, W, b):
    raise NotImplementedError("write your pallas kernel here")


# speedup vs baseline: 6.1436x; 6.1436x over previous
import functools

import jax
import jax.numpy as jnp
from jax import lax
from jax.experimental import pallas as pl
from jax.experimental.pallas import tpu as pltpu
from jax.experimental.pallas import tpu_sc as plsc

N = 10000
E = 320000
D = 128
NC = 2
NS = 16
NW = NC * NS
EPW = E // NW
K = 80
NCHUNK = EPW // K
NP = 10240
RPT = NP // NS


def _sc_body(x_hbm, src_hbm, dst_hbm, zrow_hbm, zdeg_hbm,
             part_hbm, deg_hbm,
             src_v, dst_v, rows_v, deg_v, acc_sh, sem):
    c = lax.axis_index("c")
    s = lax.axis_index("s")
    wid = c * NS + s
    ebase = wid * EPW
    rbase = s * RPT

    pltpu.sync_copy(zrow_hbm, acc_sh.at[pl.ds(rbase, RPT)])
    pltpu.sync_copy(zdeg_hbm, deg_v)
    plsc.subcore_barrier()

    one16 = jnp.ones((16,), jnp.float32)

    def chunk(i, carry):
        off = ebase + i * K
        pltpu.sync_copy(src_hbm.at[pl.ds(off, K)], src_v)
        pltpu.sync_copy(dst_hbm.at[pl.ds(off, K)], dst_v)
        pltpu.async_copy(x_hbm.at[src_v], rows_v, sem).wait()
        pltpu.sync_copy(rows_v, acc_sh.at[dst_v], add=True)
        for j in range(K // 16):
            dj = dst_v[pl.ds(j * 16, 16)]
            plsc.addupdate_scatter(deg_v, [dj], one16)
        return carry

    lax.fori_loop(0, NCHUNK, chunk, 0)
    plsc.subcore_barrier()

    pltpu.sync_copy(acc_sh.at[pl.ds(rbase, RPT)],
                    part_hbm.at[c, pl.ds(rbase, RPT)])
    pltpu.sync_copy(deg_v, deg_hbm.at[wid])


_sc_aggregate = functools.partial(
    pl.kernel,
    out_type=(
        jax.ShapeDtypeStruct((NC, NP, D), jnp.float32),
        jax.ShapeDtypeStruct((NW, NP), jnp.float32),
    ),
    mesh=plsc.VectorSubcoreMesh(core_axis_name="c", subcore_axis_name="s",
                                num_cores=NC, num_subcores=NS),
    compiler_params=pltpu.CompilerParams(needs_layout_passes=False),
    scratch_types=(
        pltpu.VMEM((K,), jnp.int32),
        pltpu.VMEM((K,), jnp.int32),
        pltpu.VMEM((K, D), jnp.float32),
        pltpu.VMEM((NP,), jnp.float32),
        pltpu.VMEM_SHARED((NP, D), jnp.float32),
        pltpu.SemaphoreType.DMA,
    ),
)(_sc_body)


def _tc_body(x_ref, p0_ref, p1_ref, deg_ref, w_ref, b_ref, o_ref):
    x = x_ref[...]
    psum = p0_ref[...] + p1_ref[...]
    dn = (((1,), (1,)), ((), ()))
    ones_w = jnp.ones((1, NW), jnp.float32)
    degcol = lax.dot_general(deg_ref[...], ones_w,
                             (((1,), (1,)), ((), ())),
                             preferred_element_type=jnp.float32)
    neigh = psum * (1.0 / jnp.maximum(degcol, 1.0))
    wl = w_ref[:, :D]
    wr = w_ref[:, D:]
    out = (lax.dot_general(x, wl, dn, preferred_element_type=jnp.float32)
           + lax.dot_general(neigh, wr, dn, preferred_element_type=jnp.float32)
           + b_ref[...])
    ss = jnp.sum(out * out, axis=1, keepdims=True)
    o_ref[...] = out / jnp.maximum(jnp.sqrt(ss), 1e-12)


BR = 1000


def _tc_finish(x, p0, p1, deg, W, b2):
    return pl.pallas_call(
        _tc_body,
        grid=(N // BR,),
        in_specs=[
            pl.BlockSpec((BR, D), lambda i: (i, 0)),
            pl.BlockSpec((BR, D), lambda i: (i, 0)),
            pl.BlockSpec((BR, D), lambda i: (i, 0)),
            pl.BlockSpec((BR, NW), lambda i: (i, 0)),
            pl.BlockSpec((D, 2 * D), lambda i: (0, 0)),
            pl.BlockSpec((1, D), lambda i: (0, 0)),
        ],
        out_specs=pl.BlockSpec((BR, D), lambda i: (i, 0)),
        out_shape=jax.ShapeDtypeStruct((N, D), jnp.float32),
    )(x, p0, p1, deg, W, b2)


def kernel(x, edge_index, W, b):
    src = edge_index[0].astype(jnp.int32)
    dst = edge_index[1].astype(jnp.int32)
    zrow = jnp.zeros((RPT, D), jnp.float32)
    zdeg = jnp.zeros((NP,), jnp.float32)
    part, deg = _sc_aggregate(x, src, dst, zrow, zdeg)
    deg_t = deg.T
    return _tc_finish(x, part[0], part[1], deg_t, W, b.reshape(1, D))
